# baseline (speedup 1.0000x reference)
import functools

import jax
import jax.numpy as jnp
from jax import lax
from jax.experimental import pallas as pl
from jax.experimental.pallas import tpu as pltpu
from jax.experimental.pallas import tpu_sc as plsc

D = 64
T = 200
B = 4096
N = B * T
NW = 32
RPW = N // NW
CH = 128
NCH = RPW // CH
NBUF = 4
LA = 3

_mesh = plsc.VectorSubcoreMesh(core_axis_name="c", subcore_axis_name="s")

SKN = 1000000
HDN = 100000
SK_FULL = SKN // 128
HD_FULL = HDN // 128
SK_NJ = (SK_FULL + NW - 1) // NW
HD_NJ = (HD_FULL + NW - 1) // NW


@functools.partial(
    pl.kernel,
    mesh=_mesh,
    out_type=(
        jax.ShapeDtypeStruct((SKN // 2, 128), jnp.float32),
        jax.ShapeDtypeStruct((HDN // 2, 128), jnp.float32),
    ),
    compiler_params=pltpu.CompilerParams(use_tc_tiling_on_sc=True,
                                         needs_layout_passes=False),
    scratch_types=[
        [pltpu.VMEM((64, 129), jnp.float32)] * 2,
        [pltpu.VMEM((64, 128), jnp.float32)] * 2,
        [pltpu.VMEM((64, 256), jnp.float32)] * 2,
        [pltpu.VMEM((128, 128), jnp.float32)] * 2,
        [pltpu.SemaphoreType.DMA] * 2,
        [pltpu.SemaphoreType.DMA] * 2,
        pltpu.SemaphoreType.DMA,
    ],
)
def _transpose_sc(sk_t_hbm, hd_t_hbm, sk_tail_hbm, hd_tail_hbm,
                  sk_out_hbm, hd_out_hbm, vbuf, obuf, vbuf2, obuf2,
                  sem_r, sem_w, sem_t):
    sid = lax.axis_index("s")
    wid = sid * 2 + lax.axis_index("c")

    iotas = [lax.iota(jnp.int32, 16) + dg * 16 for dg in range(4)]

    def transpose_block(src, dst):
        @plsc.parallel_loop(0, 128, unroll=16)
        def _(c):
            jr = c // 2
            p = c - 2 * jr
            col = jnp.full((16,), c, jnp.int32)
            for dg in range(4):
                vals = plsc.load_gather(src, [iotas[dg], col])
                dst[jr, pl.ds(p * 64 + dg * 16, 16)] = vals

    def run_table(t_hbm, out_hbm_ref, nfull, nj):
        def blk(j):
            return wid + NW * j

        def valid(j):
            return (j >= 0) & (blk(j) < nfull)

        njp = (nj + 1) // 2

        @pl.when(valid(0))
        def _():
            pltpu.async_copy(
                t_hbm.at[:, pl.ds(blk(0) * 128, 128)],
                vbuf[0].at[:, pl.ds(0, 128)], sem_r[0])

        def body(g, carry):
            for b in range(2):
                j = g * 2 + b

                @pl.when(valid(j))
                def _():
                    pltpu.make_async_copy(
                        t_hbm.at[:, pl.ds(0, 128)],
                        vbuf[b].at[:, pl.ds(0, 128)], sem_r[b]).wait()

                @pl.when(valid(j + 1))
                def _():
                    pltpu.async_copy(
                        t_hbm.at[:, pl.ds(blk(j + 1) * 128, 128)],
                        vbuf[1 - b].at[:, pl.ds(0, 128)], sem_r[1 - b])

                @pl.when(valid(j - 2))
                def _():
                    pltpu.make_async_copy(
                        obuf[b], out_hbm_ref.at[pl.ds(0, 64)], sem_w[b]).wait()

                @pl.when(valid(j))
                def _():
                    transpose_block(vbuf[b], obuf[b])
                    pltpu.async_copy(
                        obuf[b], out_hbm_ref.at[pl.ds(blk(j) * 64, 64)],
                        sem_w[b])
            return carry

        lax.fori_loop(0, njp, body, 0)

        for jj in (2 * njp - 2, 2 * njp - 1):
            @pl.when(valid(jj))
            def _():
                pltpu.make_async_copy(
                    obuf[jj % 2], out_hbm_ref.at[pl.ds(0, 64)],
                    sem_w[jj % 2]).wait()

    def transpose_block2(src, dst):
        @plsc.parallel_loop(0, 256, unroll=16)
        def _(c):
            jr = c // 2
            p = c - 2 * jr
            col = jnp.full((16,), c, jnp.int32)
            for dg in range(4):
                vals = plsc.load_gather(src, [iotas[dg], col])
                dst[jr, pl.ds(p * 64 + dg * 16, 16)] = vals

    def run_table2(t_hbm, out_hbm_ref, nfull2, nj2):
        def blk(j):
            return wid + NW * j

        def valid(j):
            return (j >= 0) & (blk(j) < nfull2)

        njp = (nj2 + 1) // 2

        @pl.when(valid(0))
        def _():
            pltpu.async_copy(
                t_hbm.at[:, pl.ds(blk(0) * 256, 256)], vbuf2[0], sem_r[0])

        def body(g, carry):
            for b in range(2):
                j = g * 2 + b

                @pl.when(valid(j))
                def _():
                    pltpu.make_async_copy(
                        t_hbm.at[:, pl.ds(0, 256)], vbuf2[b], sem_r[b]).wait()

                @pl.when(valid(j + 1))
                def _():
                    pltpu.async_copy(
                        t_hbm.at[:, pl.ds(blk(j + 1) * 256, 256)],
                        vbuf2[1 - b], sem_r[1 - b])

                @pl.when(valid(j - 2))
                def _():
                    pltpu.make_async_copy(
                        obuf2[b], out_hbm_ref.at[pl.ds(0, 128)],
                        sem_w[b]).wait()

                @pl.when(valid(j))
                def _():
                    transpose_block2(vbuf2[b], obuf2[b])
                    pltpu.async_copy(
                        obuf2[b], out_hbm_ref.at[pl.ds(blk(j) * 128, 128)],
                        sem_w[b])
            return carry

        lax.fori_loop(0, njp, body, 0)

        for jj in (2 * njp - 2, 2 * njp - 1):
            @pl.when(valid(jj))
            def _():
                pltpu.make_async_copy(
                    obuf2[jj % 2], out_hbm_ref.at[pl.ds(0, 128)],
                    sem_w[jj % 2]).wait()

    run_table2(sk_t_hbm, sk_out_hbm, SK_FULL // 2, (SK_FULL // 2 + NW - 1) // NW)
    run_table(hd_t_hbm, hd_out_hbm, HD_FULL, HD_NJ)

    @pl.when(wid == 0)
    def _():
        pltpu.sync_copy(sk_tail_hbm, vbuf[0].at[pl.ds(0, 32), pl.ds(0, 128)])
        pltpu.sync_copy(vbuf[0].at[pl.ds(0, 32), pl.ds(0, 128)],
                        sk_out_hbm.at[pl.ds(SK_FULL * 64, 32)])

    @pl.when(wid == 1)
    def _():
        pltpu.sync_copy(hd_tail_hbm, vbuf[0].at[pl.ds(0, 16), pl.ds(0, 128)])
        pltpu.sync_copy(vbuf[0].at[pl.ds(0, 16), pl.ds(0, 128)],
                        hd_out_hbm.at[pl.ds(HD_FULL * 64, 16)])


@functools.partial(
    pl.kernel,
    mesh=_mesh,
    out_type=jax.ShapeDtypeStruct((N, 2 * D), jnp.float32),
    compiler_params=pltpu.CompilerParams(use_tc_tiling_on_sc=False),
    scratch_types=[
        pltpu.VMEM((NCH, CH), jnp.int32),
        pltpu.VMEM((NCH, CH), jnp.int32),
        pltpu.VMEM_SHARED((2 * T, D), jnp.float32),
        [pltpu.VMEM((CH, D), jnp.float32)] * NBUF,
        [pltpu.SemaphoreType.DMA] * NBUF,
        [pltpu.SemaphoreType.DMA] * NBUF,
        [pltpu.SemaphoreType.DMA] * NBUF,
    ],
)
def _embed_sc(skills_hbm, hardness_hbm, pos_hbm, skill_emb_hbm, hard_emb_hbm,
              out_hbm, idx_s, idx_h, pos_sh, bufs, sem_s, sem_h, sem_o):
    sid = lax.axis_index("s")
    wid = sid * 2 + lax.axis_index("c")
    row0 = wid * RPW

    pltpu.sync_copy(skills_hbm.at[pl.ds(wid * NCH, NCH)], idx_s)
    pltpu.sync_copy(hardness_hbm.at[pl.ds(wid * NCH, NCH)], idx_h)

    @pl.when(sid == 0)
    def _():
        pltpu.sync_copy(pos_hbm, pos_sh.at[pl.ds(0, T)])
        pltpu.sync_copy(pos_hbm, pos_sh.at[pl.ds(T, T)])

    plsc.subcore_barrier()

    def init_and_gather(c, b):
        t0 = lax.rem(c * CH, T)
        pltpu.sync_copy(pos_sh.at[pl.ds(t0, CH)], bufs[b])
        pltpu.async_copy(skill_emb_hbm.at[idx_s.at[c]], bufs[b], sem_s[b],
                         add=True)
        pltpu.async_copy(hard_emb_hbm.at[idx_h.at[c]], bufs[b], sem_h[b],
                         add=True)

    for k in range(LA):
        init_and_gather(k, k)

    def body(g, carry):
        for b in range(NBUF):
            c = g * NBUF + b
            bn = (b + LA) % NBUF

            @pl.when(c + LA - NBUF >= 0)
            def _():
                pltpu.make_async_copy(
                    bufs[bn], out_hbm.at[pl.ds(row0, CH), pl.ds(0, D)], sem_o[bn]).wait()

            @pl.when(c + LA < NCH)
            def _():
                init_and_gather(c + LA, bn)

            pltpu.make_async_copy(
                skill_emb_hbm.at[idx_s.at[c]], bufs[b], sem_s[b]).wait()
            pltpu.make_async_copy(
                hard_emb_hbm.at[idx_h.at[c]], bufs[b], sem_h[b]).wait()
            pltpu.async_copy(
                bufs[b], out_hbm.at[pl.ds(row0 + c * CH, CH), pl.ds(0, D)], sem_o[b])
        return carry

    lax.fori_loop(0, NCH // NBUF, body, 0)

    bl = (NCH - 1) % NBUF
    pltpu.make_async_copy(bufs[bl], out_hbm.at[pl.ds(row0, CH), pl.ds(0, D)],
                          sem_o[bl]).wait()


def kernel(skills, hardness, position_embed, skill_embed, hardness_embed):
    skills_r = skills.reshape(N // CH, CH).astype(jnp.int32)
    hardness_r = hardness.reshape(N // CH, CH).astype(jnp.int32)
    sk_tail = skill_embed[SK_FULL * 128:].reshape(-1, 128)
    hd_tail = hardness_embed[HD_FULL * 128:].reshape(-1, 128)
    sk_packed, hd_packed = _transpose_sc(skill_embed.T, hardness_embed.T,
                                         sk_tail, hd_tail)
    out = _embed_sc(skills_r, hardness_r, position_embed,
                    sk_packed.reshape(SKN, D), hd_packed.reshape(HDN, D))
    return out[:, :D].reshape(B, T, D)


# speedup vs baseline: 1.2422x; 1.2422x over previous
import functools

import jax
import jax.numpy as jnp
from jax import lax
from jax.experimental import pallas as pl
from jax.experimental.pallas import tpu as pltpu
from jax.experimental.pallas import tpu_sc as plsc

D = 64
T = 200
B = 4096
N = B * T
NW = 32
RPW = N // NW
CH = 128
NCH = RPW // CH
NBUF = 4
LA = 3

_mesh = plsc.VectorSubcoreMesh(core_axis_name="c", subcore_axis_name="s")


@functools.partial(
    pl.kernel,
    mesh=_mesh,
    out_type=jax.ShapeDtypeStruct((N, 2 * D), jnp.float32),
    compiler_params=pltpu.CompilerParams(use_tc_tiling_on_sc=False),
    scratch_types=[
        pltpu.VMEM((NCH, CH), jnp.int32),
        pltpu.VMEM((NCH, CH), jnp.int32),
        pltpu.VMEM_SHARED((2 * T, D), jnp.float32),
        [pltpu.VMEM((CH, D), jnp.float32)] * NBUF,
        [pltpu.SemaphoreType.DMA] * NBUF,
        [pltpu.SemaphoreType.DMA] * NBUF,
        [pltpu.SemaphoreType.DMA] * NBUF,
    ],
)
def _embed_sc(skills_hbm, hardness_hbm, pos_hbm, skill_emb_hbm, hard_emb_hbm,
              out_hbm, idx_s, idx_h, pos_sh, bufs, sem_s, sem_h, sem_o):
    sid = lax.axis_index("s")
    wid = sid * 2 + lax.axis_index("c")
    row0 = wid * RPW

    pltpu.sync_copy(skills_hbm.at[pl.ds(wid * NCH, NCH)], idx_s)
    pltpu.sync_copy(hardness_hbm.at[pl.ds(wid * NCH, NCH)], idx_h)

    @pl.when(sid == 0)
    def _():
        pltpu.sync_copy(pos_hbm, pos_sh.at[pl.ds(0, T)])
        pltpu.sync_copy(pos_hbm, pos_sh.at[pl.ds(T, T)])

    plsc.subcore_barrier()

    def init_and_gather(c, b):
        t0 = lax.rem(c * CH, T)
        pltpu.sync_copy(pos_sh.at[pl.ds(t0, CH)], bufs[b])
        pltpu.async_copy(skill_emb_hbm.at[idx_s.at[c]], bufs[b], sem_s[b],
                         add=True)
        pltpu.async_copy(hard_emb_hbm.at[idx_h.at[c]], bufs[b], sem_h[b],
                         add=True)

    for k in range(LA):
        init_and_gather(k, k)

    def body(g, carry):
        for b in range(NBUF):
            c = g * NBUF + b
            bn = (b + LA) % NBUF

            @pl.when(c + LA - NBUF >= 0)
            def _():
                pltpu.make_async_copy(
                    bufs[bn], out_hbm.at[pl.ds(row0, CH), pl.ds(0, D)], sem_o[bn]).wait()

            @pl.when(c + LA < NCH)
            def _():
                init_and_gather(c + LA, bn)

            pltpu.make_async_copy(
                skill_emb_hbm.at[idx_s.at[c]], bufs[b], sem_s[b]).wait()
            pltpu.make_async_copy(
                hard_emb_hbm.at[idx_h.at[c]], bufs[b], sem_h[b]).wait()
            pltpu.async_copy(
                bufs[b], out_hbm.at[pl.ds(row0 + c * CH, CH), pl.ds(0, D)], sem_o[b])
        return carry

    lax.fori_loop(0, NCH // NBUF, body, 0)

    bl = (NCH - 1) % NBUF
    pltpu.make_async_copy(bufs[bl], out_hbm.at[pl.ds(row0, CH), pl.ds(0, D)],
                          sem_o[bl]).wait()


def kernel(skills, hardness, position_embed, skill_embed, hardness_embed):
    skills_r = skills.reshape(N // CH, CH).astype(jnp.int32)
    hardness_r = hardness.reshape(N // CH, CH).astype(jnp.int32)
    out = _embed_sc(skills_r, hardness_r, position_embed, skill_embed,
                    hardness_embed)
    return out[:, :D].reshape(B, T, D)
